# Initial kernel scaffold; baseline (speedup 1.0000x reference)
import jax
import jax.numpy as jnp
from jax.experimental import pallas as pl


def kernel(x, labels, centers):
    raise NotImplementedError("write your pallas kernel here")


# speedup vs baseline: 2.8837x; 2.8837x over previous
import jax
import jax.numpy as jnp
from jax.experimental import pallas as pl
from jax.experimental.pallas import tpu as pltpu

_N = 16384
_D = 1024
_C = 1000
_CP = 1024
_B = 512
_GRID = _N // _B


def _center_loss_kernel(lab_ref, x_ref, cen_ref, out_ref, s_acc, a_acc):
    i = pl.program_id(0)

    @pl.when(i == 0)
    def _init():
        s_acc[...] = jnp.zeros_like(s_acc)
        a_acc[...] = jnp.zeros_like(a_acc)

    x = x_ref[...]
    lab = lab_ref[0]
    onehot_t = (jax.lax.broadcasted_iota(jnp.int32, (_CP, _B), 0) == lab
                ).astype(jnp.float32)

    s_acc[...] += jax.lax.dot_general(
        onehot_t, x, (((1,), (0,)), ((), ())),
        preferred_element_type=jnp.float32)

    d2row = jnp.sum(x * x, axis=1, keepdims=True)
    lane = jax.lax.broadcasted_iota(jnp.int32, (_B, 128), 1)
    aux = jnp.where(lane == 0, d2row,
                    jnp.where(lane == 1, 1.0, 0.0))
    a_acc[...] += jax.lax.dot_general(
        onehot_t, aux, (((1,), (0,)), ((), ())),
        preferred_element_type=jnp.float32)

    @pl.when(i == _GRID - 1)
    def _epilogue():
        cen = cen_ref[...]
        s = s_acc[...]
        t = a_acc[..., 0:1]
        n = a_acc[..., 1:2]
        sc = jnp.sum(s * cen, axis=1, keepdims=True)
        cn = jnp.sum(cen * cen, axis=1, keepdims=True)
        pc = t - 2.0 * sc + n * cn
        norms = jnp.where(pc > 0, jnp.sqrt(jnp.where(pc > 0, pc, 1.0)), 0.0)
        out_ref[...] = jnp.sum(norms, keepdims=True) / _C


def kernel(x, labels, centers):
    lab3 = labels.reshape(_GRID, 1, _B)
    cen_pad = jnp.pad(centers, ((0, _CP - _C), (0, 0)))
    out = pl.pallas_call(
        _center_loss_kernel,
        grid=(_GRID,),
        in_specs=[
            pl.BlockSpec((1, 1, _B), lambda i: (i, 0, 0)),
            pl.BlockSpec((_B, _D), lambda i: (i, 0)),
            pl.BlockSpec((_CP, _D), lambda i: (0, 0)),
        ],
        out_specs=pl.BlockSpec((1, 1), lambda i: (0, 0)),
        out_shape=jax.ShapeDtypeStruct((1, 1), jnp.float32),
        scratch_shapes=[
            pltpu.VMEM((_CP, _D), jnp.float32),
            pltpu.VMEM((_CP, 128), jnp.float32),
        ],
        compiler_params=pltpu.CompilerParams(
            dimension_semantics=("arbitrary",)),
    )(lab3, x, cen_pad)
    return out[0, 0]


# speedup vs baseline: 2.8868x; 1.0011x over previous
import jax
import jax.numpy as jnp
from jax.experimental import pallas as pl
from jax.experimental.pallas import tpu as pltpu

_N = 16384
_D = 1024
_C = 1000
_CP = 1024
_B = 512
_GRID = _N // _B


def _center_loss_kernel(lab_ref, x_ref, cen_ref, out_ref, s_acc, a_acc):
    i = pl.program_id(0)

    @pl.when(i == 0)
    def _init():
        s_acc[...] = jnp.zeros_like(s_acc)
        a_acc[...] = jnp.zeros_like(a_acc)

    x = x_ref[...]
    lab = lab_ref[0]
    onehot_t = (jax.lax.broadcasted_iota(jnp.int32, (_CP, _B), 0) == lab
                ).astype(jnp.float32)

    s_acc[...] += jax.lax.dot_general(
        onehot_t.astype(jnp.bfloat16), x.astype(jnp.bfloat16),
        (((1,), (0,)), ((), ())),
        preferred_element_type=jnp.float32)

    d2row = jnp.sum(x * x, axis=1, keepdims=True)
    lane = jax.lax.broadcasted_iota(jnp.int32, (_B, 128), 1)
    aux = jnp.where(lane == 0, d2row,
                    jnp.where(lane == 1, 1.0, 0.0))
    a_acc[...] += jax.lax.dot_general(
        onehot_t, aux, (((1,), (0,)), ((), ())),
        preferred_element_type=jnp.float32)

    @pl.when(i == _GRID - 1)
    def _epilogue():
        cen = cen_ref[...]
        s = s_acc[...]
        t = a_acc[..., 0:1]
        n = a_acc[..., 1:2]
        sc = jnp.sum(s * cen, axis=1, keepdims=True)
        cn = jnp.sum(cen * cen, axis=1, keepdims=True)
        pc = t - 2.0 * sc + n * cn
        norms = jnp.where(pc > 0, jnp.sqrt(jnp.where(pc > 0, pc, 1.0)), 0.0)
        out_ref[...] = jnp.sum(norms, keepdims=True) / _C


def kernel(x, labels, centers):
    lab3 = labels.reshape(_GRID, 1, _B)
    cen_pad = jnp.pad(centers, ((0, _CP - _C), (0, 0)))
    out = pl.pallas_call(
        _center_loss_kernel,
        grid=(_GRID,),
        in_specs=[
            pl.BlockSpec((1, 1, _B), lambda i: (i, 0, 0)),
            pl.BlockSpec((_B, _D), lambda i: (i, 0)),
            pl.BlockSpec((_CP, _D), lambda i: (0, 0)),
        ],
        out_specs=pl.BlockSpec((1, 1), lambda i: (0, 0)),
        out_shape=jax.ShapeDtypeStruct((1, 1), jnp.float32),
        scratch_shapes=[
            pltpu.VMEM((_CP, _D), jnp.float32),
            pltpu.VMEM((_CP, 128), jnp.float32),
        ],
        compiler_params=pltpu.CompilerParams(
            dimension_semantics=("arbitrary",)),
    )(lab3, x, cen_pad)
    return out[0, 0]
